# baseline (speedup 1.0000x reference)
import jax
import jax.numpy as jnp
from jax import lax
from jax.experimental import pallas as pl
from jax.experimental.pallas import tpu as pltpu
from jax.experimental.pallas import tpu_sc as plsc

B, S, H = 4, 2048, 768
L = 16
NCH = H // L
NC, NS = 2, 16
NW = NC * NS
POS_PER_W = S // NW
TP = 4
TT = TP * B
NTILES = POS_PER_W // TP
NT2 = NTILES // 2
EPS = 1e-12


def _rsqrt_newton(x):
    i = lax.bitcast_convert_type(x, jnp.int32)
    i = jnp.int32(0x5F3759DF) - lax.shift_right_logical(i, 1)
    y = lax.bitcast_convert_type(i, jnp.float32)
    for _ in range(4):
        y = y * (jnp.float32(1.5) - jnp.float32(0.5) * x * y * y)
    return y


def _lane_sum(x):
    idx0 = lax.iota(jnp.int32, L)
    for sh in (8, 4, 2, 1):
        x = x + x.at[idx0 ^ sh].get(mode="promise_in_bounds")
    return x


def _body(ids_t, sids_t, pids_t, word, shape_t, pron, pos, tt, gamma, beta,
          out,
          idxw0, idxs0, idxp0, idxw1, idxs1, idxp1, oidx,
          wv0, sv0, pv0, wv1, sv1, pv1, outv, posv0, posv1, ttv, gv, bv,
          semw0, sems0, semp0, semw1, sems1, semp1):
    cid = lax.axis_index("c")
    sid = lax.axis_index("s")
    wid = sid * NC + cid
    pos0 = wid * POS_PER_W
    third = jnp.float32(1.0 / 3.0)

    pltpu.sync_copy(tt.at[0], ttv)
    pltpu.sync_copy(gamma, gv)
    pltpu.sync_copy(beta, bv)
    for i in range(NCH):
        bv[pl.ds(i * L, L)] = bv[pl.ds(i * L, L)] * third

    bufs = (
        (idxw0, idxs0, idxp0, posv0, wv0, sv0, pv0, semw0, sems0, semp0),
        (idxw1, idxs1, idxp1, posv1, wv1, sv1, pv1, semw1, sems1, semp1),
    )

    def launch(t, bf):
        idxw, idxs, idxp, posv, wv, sv, pv, semw, sems, semp = bf
        pbase = pos0 + t * TP
        tbase = pbase * B
        pltpu.sync_copy(ids_t.at[pl.ds(tbase, TT)], idxw)
        pltpu.sync_copy(sids_t.at[pl.ds(tbase, TT)], idxs)
        pltpu.sync_copy(pids_t.at[pl.ds(tbase, TT)], idxp)
        pltpu.sync_copy(pos.at[pl.ds(pbase, TP)], posv)
        pltpu.async_copy(word.at[idxw], wv, semw)
        pltpu.async_copy(shape_t.at[idxs], sv, sems)
        pltpu.async_copy(pron.at[idxp], pv, semp)

    def finish(t, bf):
        idxw, idxs, idxp, posv, wv, sv, pv, semw, sems, semp = bf
        pbase = pos0 + t * TP

        pltpu.make_async_copy(word.at[idxw], wv, semw).wait()
        pltpu.make_async_copy(shape_t.at[idxs], sv, sems).wait()
        pltpu.make_async_copy(pron.at[idxp], pv, semp).wait()

        def fold(jp, _):
            for i in range(NCH):
                d = pl.ds(i * L, L)
                posv[jp, d] = posv[jp, d] + ttv[d]
            return 0
        lax.fori_loop(0, TP, fold, 0)

        k = lax.iota(jnp.int32, L)
        oidx[...] = (k & jnp.int32(B - 1)) * jnp.int32(S) \
            + pbase + lax.shift_right_logical(k, 2)

        z = jnp.zeros((L,), jnp.float32)

        HG = TT // 4
        lanes = lax.iota(jnp.int32, L)
        s_v, q_v = z, z
        for half in range(4):
            toks = list(range(half * HG, (half + 1) * HG))
            jps = sorted({j >> 2 for j in toks})

            def ch_acc(i, carry, toks=toks, jps=jps):
                d = pl.ds(i * L, L)
                pt = {jp: posv[jp, d] for jp in jps}
                acc = []
                for n, j in enumerate(toks):
                    v = wv[j, d] + pt[j >> 2]
                    acc.append(carry[n] + v)
                    acc.append(carry[HG + n] + v * v)
                return tuple(acc[0::2] + acc[1::2])

            accs = lax.fori_loop(0, NCH, ch_acc, tuple([z] * (2 * HG)))

            for n, j in enumerate(toks):
                sel = lanes == jnp.int32(j)
                s_v = jnp.where(sel, _lane_sum(accs[n]), s_v)
                q_v = jnp.where(sel, _lane_sum(accs[HG + n]), q_v)

        mean_v = s_v * jnp.float32(1.0 / H)
        var_v = q_v * jnp.float32(1.0 / H) - mean_v * mean_v
        a3_v = _rsqrt_newton(var_v + jnp.float32(EPS)) * third

        for half in range(4):
            toks = list(range(half * HG, (half + 1) * HG))
            jps = sorted({j >> 2 for j in toks})
            means = {}
            a3s = {}
            for j in toks:
                jsplat = jnp.full((L,), j, jnp.int32)
                means[j] = mean_v.at[jsplat].get(mode="promise_in_bounds")
                a3s[j] = a3_v.at[jsplat].get(mode="promise_in_bounds")

            def ch_out(i, _, toks=toks, jps=jps, means=means, a3s=a3s):
                d = pl.ds(i * L, L)
                g_ = gv[d]
                b3_ = bv[d]
                pt = {jp: posv[jp, d] for jp in jps}
                for j in toks:
                    v = wv[j, d] + pt[j >> 2]
                    o = (v - means[j]) * a3s[j] * g_ \
                        + (b3_ + (sv[j, d] + pv[j, d]) * third)
                    outv[j, d] = o
                return 0
            lax.fori_loop(0, NCH, ch_out, 0)

        pltpu.sync_copy(outv, out.at[oidx])

    launch(0, bufs[0])

    def pipe(i, _):
        t0 = 2 * i
        launch(t0 + 1, bufs[1])
        finish(t0, bufs[0])

        @pl.when(i < NT2 - 1)
        def _():
            launch(t0 + 2, bufs[0])

        finish(t0 + 1, bufs[1])
        return 0

    lax.fori_loop(0, NT2, pipe, 0)


@jax.jit
def _sc_call(ids_t, sids_t, pids_t, word, shape_t, pron, pos, tt, gamma, beta):
    mesh = plsc.VectorSubcoreMesh(core_axis_name="c", subcore_axis_name="s",
                                  num_cores=NC, num_subcores=NS)
    f = pl.kernel(
        _body,
        out_type=jax.ShapeDtypeStruct((B * S, H), jnp.float32),
        mesh=mesh,
        scratch_types=[
            pltpu.VMEM((TT,), jnp.int32),
            pltpu.VMEM((TT,), jnp.int32),
            pltpu.VMEM((TT,), jnp.int32),
            pltpu.VMEM((TT,), jnp.int32),
            pltpu.VMEM((TT,), jnp.int32),
            pltpu.VMEM((TT,), jnp.int32),
            pltpu.VMEM((TT,), jnp.int32),
            pltpu.VMEM((TT, H), jnp.float32),
            pltpu.VMEM((TT, H), jnp.float32),
            pltpu.VMEM((TT, H), jnp.float32),
            pltpu.VMEM((TT, H), jnp.float32),
            pltpu.VMEM((TT, H), jnp.float32),
            pltpu.VMEM((TT, H), jnp.float32),
            pltpu.VMEM((TT, H), jnp.float32),
            pltpu.VMEM((TP, H), jnp.float32),
            pltpu.VMEM((TP, H), jnp.float32),
            pltpu.VMEM((H,), jnp.float32),
            pltpu.VMEM((H,), jnp.float32),
            pltpu.VMEM((H,), jnp.float32),
            pltpu.SemaphoreType.DMA,
            pltpu.SemaphoreType.DMA,
            pltpu.SemaphoreType.DMA,
            pltpu.SemaphoreType.DMA,
            pltpu.SemaphoreType.DMA,
            pltpu.SemaphoreType.DMA,
        ],
    )
    return f(ids_t, sids_t, pids_t, word, shape_t, pron, pos, tt, gamma, beta)


def kernel(input_ids, input_shape_ids, input_pronunciation_ids,
           word_embeddings, shape_embed, pronunciation_embed,
           position_embeddings, token_type_embeddings, ln_weight, ln_bias):
    ids_t = input_ids.astype(jnp.int32).T.reshape(-1)
    sids_t = input_shape_ids.astype(jnp.int32).T.reshape(-1)
    pids_t = input_pronunciation_ids.astype(jnp.int32).T.reshape(-1)
    out = _sc_call(ids_t, sids_t, pids_t, word_embeddings, shape_embed,
                   pronunciation_embed, position_embeddings,
                   token_type_embeddings, ln_weight, ln_bias)
    return out.reshape(B, S, H)


# speedup vs baseline: 1.0440x; 1.0440x over previous
import jax
import jax.numpy as jnp
from jax import lax
from jax.experimental import pallas as pl
from jax.experimental.pallas import tpu as pltpu
from jax.experimental.pallas import tpu_sc as plsc

B, S, H = 4, 2048, 768
L = 16
NCH = H // L
NC, NS = 2, 16
NW = NC * NS
POS_PER_W = S // NW
TP = 4
TT = TP * B
NTILES = POS_PER_W // TP
NT2 = NTILES // 2
EPS = 1e-12


def _rsqrt_newton(x):
    i = lax.bitcast_convert_type(x, jnp.int32)
    i = jnp.int32(0x5F3759DF) - lax.shift_right_logical(i, 1)
    y = lax.bitcast_convert_type(i, jnp.float32)
    for _ in range(4):
        y = y * (jnp.float32(1.5) - jnp.float32(0.5) * x * y * y)
    return y


def _lane_sum(x):
    idx0 = lax.iota(jnp.int32, L)
    for sh in (8, 4, 2, 1):
        x = x + x.at[idx0 ^ sh].get(mode="promise_in_bounds")
    return x


def _body(ids_t, sids_t, pids_t, word, shape_t, pron, pos, tt, gamma, beta,
          out,
          idxw0, idxs0, idxp0, idxw1, idxs1, idxp1, oidx,
          wv0, sv0, pv0, wv1, sv1, pv1, outv, posv0, posv1, ttv, gv, bv,
          semw0, sems0, semp0, semw1, sems1, semp1):
    cid = lax.axis_index("c")
    sid = lax.axis_index("s")
    wid = sid * NC + cid
    pos0 = wid * POS_PER_W
    third = jnp.float32(1.0 / 3.0)

    pltpu.sync_copy(tt.at[0], ttv)
    pltpu.sync_copy(gamma, gv)
    pltpu.sync_copy(beta, bv)
    for i in range(NCH):
        bv[pl.ds(i * L, L)] = bv[pl.ds(i * L, L)] * third

    bufs = (
        (idxw0, idxs0, idxp0, posv0, wv0, sv0, pv0, semw0, sems0, semp0),
        (idxw1, idxs1, idxp1, posv1, wv1, sv1, pv1, semw1, sems1, semp1),
    )

    def launch(t, bf):
        idxw, idxs, idxp, posv, wv, sv, pv, semw, sems, semp = bf
        pbase = pos0 + t * TP
        tbase = pbase * B
        pltpu.sync_copy(ids_t.at[pl.ds(tbase, TT)], idxw)
        pltpu.sync_copy(sids_t.at[pl.ds(tbase, TT)], idxs)
        pltpu.sync_copy(pids_t.at[pl.ds(tbase, TT)], idxp)
        pltpu.sync_copy(pos.at[pl.ds(pbase, TP)], posv)
        pltpu.async_copy(word.at[idxw], wv, semw)
        pltpu.async_copy(shape_t.at[idxs], sv, sems)
        pltpu.async_copy(pron.at[idxp], pv, semp)

    def finish(t, bf):
        idxw, idxs, idxp, posv, wv, sv, pv, semw, sems, semp = bf
        pbase = pos0 + t * TP

        pltpu.make_async_copy(word.at[idxw], wv, semw).wait()
        pltpu.make_async_copy(shape_t.at[idxs], sv, sems).wait()
        pltpu.make_async_copy(pron.at[idxp], pv, semp).wait()

        def fold(jp, _):
            for i in range(NCH):
                d = pl.ds(i * L, L)
                posv[jp, d] = posv[jp, d] + ttv[d]
            return 0
        lax.fori_loop(0, TP, fold, 0)

        k = lax.iota(jnp.int32, L)
        oidx[...] = (k & jnp.int32(B - 1)) * jnp.int32(S) \
            + pbase + lax.shift_right_logical(k, 2)

        z = jnp.zeros((L,), jnp.float32)

        HG = TT // 2
        lanes = lax.iota(jnp.int32, L)
        s_v, q_v = z, z
        for half in range(2):
            toks = list(range(half * HG, (half + 1) * HG))
            jps = sorted({j >> 2 for j in toks})

            def ch_acc(i, carry, toks=toks, jps=jps):
                d = pl.ds(i * L, L)
                pt = {jp: posv[jp, d] for jp in jps}
                acc = []
                for n, j in enumerate(toks):
                    v = wv[j, d] + pt[j >> 2]
                    wv[j, d] = v
                    acc.append(carry[n] + v)
                    acc.append(carry[HG + n] + v * v)
                return tuple(acc[0::2] + acc[1::2])

            accs = lax.fori_loop(0, NCH, ch_acc, tuple([z] * (2 * HG)))

            for n, j in enumerate(toks):
                sel = lanes == jnp.int32(j)
                s_v = jnp.where(sel, _lane_sum(accs[n]), s_v)
                q_v = jnp.where(sel, _lane_sum(accs[HG + n]), q_v)

        mean_v = s_v * jnp.float32(1.0 / H)
        var_v = q_v * jnp.float32(1.0 / H) - mean_v * mean_v
        a3_v = _rsqrt_newton(var_v + jnp.float32(EPS)) * third

        for half in range(2):
            toks = list(range(half * HG, (half + 1) * HG))
            jps = sorted({j >> 2 for j in toks})
            means = {}
            a3s = {}
            for j in toks:
                jsplat = jnp.full((L,), j, jnp.int32)
                means[j] = mean_v.at[jsplat].get(mode="promise_in_bounds")
                a3s[j] = a3_v.at[jsplat].get(mode="promise_in_bounds")

            def ch_out(i, _, toks=toks, jps=jps, means=means, a3s=a3s):
                d = pl.ds(i * L, L)
                g_ = gv[d]
                b3_ = bv[d]
                for j in toks:
                    v = wv[j, d]
                    o = (v - means[j]) * a3s[j] * g_ \
                        + (b3_ + (sv[j, d] + pv[j, d]) * third)
                    outv[j, d] = o
                return 0
            lax.fori_loop(0, NCH, ch_out, 0)

        pltpu.sync_copy(outv, out.at[oidx])

    launch(0, bufs[0])

    def pipe(i, _):
        t0 = 2 * i
        launch(t0 + 1, bufs[1])
        finish(t0, bufs[0])

        @pl.when(i < NT2 - 1)
        def _():
            launch(t0 + 2, bufs[0])

        finish(t0 + 1, bufs[1])
        return 0

    lax.fori_loop(0, NT2, pipe, 0)


@jax.jit
def _sc_call(ids_t, sids_t, pids_t, word, shape_t, pron, pos, tt, gamma, beta):
    mesh = plsc.VectorSubcoreMesh(core_axis_name="c", subcore_axis_name="s",
                                  num_cores=NC, num_subcores=NS)
    f = pl.kernel(
        _body,
        out_type=jax.ShapeDtypeStruct((B * S, H), jnp.float32),
        mesh=mesh,
        scratch_types=[
            pltpu.VMEM((TT,), jnp.int32),
            pltpu.VMEM((TT,), jnp.int32),
            pltpu.VMEM((TT,), jnp.int32),
            pltpu.VMEM((TT,), jnp.int32),
            pltpu.VMEM((TT,), jnp.int32),
            pltpu.VMEM((TT,), jnp.int32),
            pltpu.VMEM((TT,), jnp.int32),
            pltpu.VMEM((TT, H), jnp.float32),
            pltpu.VMEM((TT, H), jnp.float32),
            pltpu.VMEM((TT, H), jnp.float32),
            pltpu.VMEM((TT, H), jnp.float32),
            pltpu.VMEM((TT, H), jnp.float32),
            pltpu.VMEM((TT, H), jnp.float32),
            pltpu.VMEM((TT, H), jnp.float32),
            pltpu.VMEM((TP, H), jnp.float32),
            pltpu.VMEM((TP, H), jnp.float32),
            pltpu.VMEM((H,), jnp.float32),
            pltpu.VMEM((H,), jnp.float32),
            pltpu.VMEM((H,), jnp.float32),
            pltpu.SemaphoreType.DMA,
            pltpu.SemaphoreType.DMA,
            pltpu.SemaphoreType.DMA,
            pltpu.SemaphoreType.DMA,
            pltpu.SemaphoreType.DMA,
            pltpu.SemaphoreType.DMA,
        ],
    )
    return f(ids_t, sids_t, pids_t, word, shape_t, pron, pos, tt, gamma, beta)


def kernel(input_ids, input_shape_ids, input_pronunciation_ids,
           word_embeddings, shape_embed, pronunciation_embed,
           position_embeddings, token_type_embeddings, ln_weight, ln_bias):
    ids_t = input_ids.astype(jnp.int32).T.reshape(-1)
    sids_t = input_shape_ids.astype(jnp.int32).T.reshape(-1)
    pids_t = input_pronunciation_ids.astype(jnp.int32).T.reshape(-1)
    out = _sc_call(ids_t, sids_t, pids_t, word_embeddings, shape_embed,
                   pronunciation_embed, position_embeddings,
                   token_type_embeddings, ln_weight, ln_bias)
    return out.reshape(B, S, H)


# speedup vs baseline: 1.2037x; 1.1529x over previous
import jax
import jax.numpy as jnp
from jax import lax
from jax.experimental import pallas as pl
from jax.experimental.pallas import tpu as pltpu
from jax.experimental.pallas import tpu_sc as plsc

B, S, H = 4, 2048, 768
L = 16
NCH = H // L
NC, NS = 2, 16
NW = NC * NS
POS_PER_W = S // NW
TP = 4
TT = TP * B
NTILES = POS_PER_W // TP
NT2 = NTILES // 2
EPS = 1e-12


def _rsqrt_newton(x):
    i = lax.bitcast_convert_type(x, jnp.int32)
    i = jnp.int32(0x5F3759DF) - lax.shift_right_logical(i, 1)
    y = lax.bitcast_convert_type(i, jnp.float32)
    for _ in range(4):
        y = y * (jnp.float32(1.5) - jnp.float32(0.5) * x * y * y)
    return y


def _lane_sum(x):
    idx0 = lax.iota(jnp.int32, L)
    for sh in (8, 4, 2, 1):
        x = x + x.at[idx0 ^ sh].get(mode="promise_in_bounds")
    return x


def _body(ids2, sids2, pids2, word, shape_t, pron, pos, tt, gamma, beta,
          out,
          idxw, idxs, idxp, oidx,
          wv0, sv0, pv0, wv1, sv1, pv1, outv, posv0, posv1, ttv, gv, bv,
          semw0, sems0, semp0, semw1, sems1, semp1):
    cid = lax.axis_index("c")
    sid = lax.axis_index("s")
    wid = sid * NC + cid
    pos0 = wid * POS_PER_W
    third = jnp.float32(1.0 / 3.0)

    pltpu.sync_copy(tt.at[0], ttv)
    pltpu.sync_copy(gamma, gv)
    pltpu.sync_copy(beta, bv)
    row0 = wid * NTILES
    pltpu.sync_copy(ids2.at[pl.ds(row0, NTILES)], idxw)
    pltpu.sync_copy(sids2.at[pl.ds(row0, NTILES)], idxs)
    pltpu.sync_copy(pids2.at[pl.ds(row0, NTILES)], idxp)
    for i in range(NCH):
        bv[pl.ds(i * L, L)] = bv[pl.ds(i * L, L)] * third

    bufs = (
        (posv0, wv0, sv0, pv0, semw0, sems0, semp0),
        (posv1, wv1, sv1, pv1, semw1, sems1, semp1),
    )

    def launch(t, bf):
        posv, wv, sv, pv, semw, sems, semp = bf
        pbase = pos0 + t * TP
        pltpu.sync_copy(pos.at[pl.ds(pbase, TP)], posv)
        pltpu.async_copy(word.at[idxw.at[t]], wv, semw)
        pltpu.async_copy(shape_t.at[idxs.at[t]], sv, sems)
        pltpu.async_copy(pron.at[idxp.at[t]], pv, semp)

    def finish(t, bf):
        posv, wv, sv, pv, semw, sems, semp = bf
        pbase = pos0 + t * TP

        def fold(jp, _):
            for i in range(NCH):
                d = pl.ds(i * L, L)
                posv[jp, d] = posv[jp, d] + ttv[d]
            return 0
        lax.fori_loop(0, TP, fold, 0)
        k = lax.iota(jnp.int32, L)
        oidx[...] = (k & jnp.int32(B - 1)) * jnp.int32(S) \
            + pbase + lax.shift_right_logical(k, 2)

        pltpu.make_async_copy(word.at[idxw.at[t]], wv, semw).wait()
        pltpu.make_async_copy(shape_t.at[idxs.at[t]], sv, sems).wait()
        pltpu.make_async_copy(pron.at[idxp.at[t]], pv, semp).wait()

        z = jnp.zeros((L,), jnp.float32)

        HG = TT // 2
        lanes = lax.iota(jnp.int32, L)
        s_v, q_v = z, z
        for half in range(2):
            toks = list(range(half * HG, (half + 1) * HG))
            jps = sorted({j >> 2 for j in toks})

            def ch_acc(i, carry, toks=toks, jps=jps):
                d = pl.ds(i * L, L)
                pt = {jp: posv[jp, d] for jp in jps}
                acc = []
                for n, j in enumerate(toks):
                    v = wv[j, d] + pt[j >> 2]
                    acc.append(carry[n] + v)
                    acc.append(carry[HG + n] + v * v)
                return tuple(acc[0::2] + acc[1::2])

            accs = lax.fori_loop(0, NCH, ch_acc, tuple([z] * (2 * HG)))

            for n, j in enumerate(toks):
                sel = lanes == jnp.int32(j)
                s_v = jnp.where(sel, _lane_sum(accs[n]), s_v)
                q_v = jnp.where(sel, _lane_sum(accs[HG + n]), q_v)

        mean_v = s_v * jnp.float32(1.0 / H)
        var_v = q_v * jnp.float32(1.0 / H) - mean_v * mean_v
        a3_v = _rsqrt_newton(var_v + jnp.float32(EPS)) * third

        for half in range(2):
            toks = list(range(half * HG, (half + 1) * HG))
            jps = sorted({j >> 2 for j in toks})
            means = {}
            a3s = {}
            for j in toks:
                jsplat = jnp.full((L,), j, jnp.int32)
                means[j] = mean_v.at[jsplat].get(mode="promise_in_bounds")
                a3s[j] = a3_v.at[jsplat].get(mode="promise_in_bounds")

            def ch_out(i, _, toks=toks, jps=jps, means=means, a3s=a3s):
                d = pl.ds(i * L, L)
                g_ = gv[d]
                b3_ = bv[d]
                pt = {jp: posv[jp, d] for jp in jps}
                for j in toks:
                    v = wv[j, d] + pt[j >> 2]
                    o = (v - means[j]) * a3s[j] * g_ \
                        + (b3_ + (sv[j, d] + pv[j, d]) * third)
                    outv[j, d] = o
                return 0
            lax.fori_loop(0, NCH, ch_out, 0)

        pltpu.sync_copy(outv, out.at[oidx])

    launch(0, bufs[0])

    def pipe(i, _):
        t0 = 2 * i
        launch(t0 + 1, bufs[1])
        finish(t0, bufs[0])

        @pl.when(i < NT2 - 1)
        def _():
            launch(t0 + 2, bufs[0])

        finish(t0 + 1, bufs[1])
        return 0

    lax.fori_loop(0, NT2, pipe, 0)


@jax.jit
def _sc_call(ids_t, sids_t, pids_t, word, shape_t, pron, pos, tt, gamma, beta):
    mesh = plsc.VectorSubcoreMesh(core_axis_name="c", subcore_axis_name="s",
                                  num_cores=NC, num_subcores=NS)
    f = pl.kernel(
        _body,
        out_type=jax.ShapeDtypeStruct((B * S, H), jnp.float32),
        mesh=mesh,
        scratch_types=[
            pltpu.VMEM((NTILES, TT), jnp.int32),
            pltpu.VMEM((NTILES, TT), jnp.int32),
            pltpu.VMEM((NTILES, TT), jnp.int32),
            pltpu.VMEM((TT,), jnp.int32),
            pltpu.VMEM((TT, H), jnp.float32),
            pltpu.VMEM((TT, H), jnp.float32),
            pltpu.VMEM((TT, H), jnp.float32),
            pltpu.VMEM((TT, H), jnp.float32),
            pltpu.VMEM((TT, H), jnp.float32),
            pltpu.VMEM((TT, H), jnp.float32),
            pltpu.VMEM((TT, H), jnp.float32),
            pltpu.VMEM((TP, H), jnp.float32),
            pltpu.VMEM((TP, H), jnp.float32),
            pltpu.VMEM((H,), jnp.float32),
            pltpu.VMEM((H,), jnp.float32),
            pltpu.VMEM((H,), jnp.float32),
            pltpu.SemaphoreType.DMA,
            pltpu.SemaphoreType.DMA,
            pltpu.SemaphoreType.DMA,
            pltpu.SemaphoreType.DMA,
            pltpu.SemaphoreType.DMA,
            pltpu.SemaphoreType.DMA,
        ],
    )
    return f(ids_t, sids_t, pids_t, word, shape_t, pron, pos, tt, gamma, beta)


def kernel(input_ids, input_shape_ids, input_pronunciation_ids,
           word_embeddings, shape_embed, pronunciation_embed,
           position_embeddings, token_type_embeddings, ln_weight, ln_bias):
    ids_t = input_ids.astype(jnp.int32).T.reshape(-1, TT)
    sids_t = input_shape_ids.astype(jnp.int32).T.reshape(-1, TT)
    pids_t = input_pronunciation_ids.astype(jnp.int32).T.reshape(-1, TT)
    out = _sc_call(ids_t, sids_t, pids_t, word_embeddings, shape_embed,
                   pronunciation_embed, position_embeddings,
                   token_type_embeddings, ln_weight, ln_bias)
    return out.reshape(B, S, H)


# speedup vs baseline: 1.4437x; 1.1994x over previous
import jax
import jax.numpy as jnp
from jax import lax
from jax.experimental import pallas as pl
from jax.experimental.pallas import tpu as pltpu
from jax.experimental.pallas import tpu_sc as plsc

B, S, H = 4, 2048, 768
L = 16
NCH = H // L
NC, NS = 2, 16
NW = NC * NS
POS_PER_W = S // NW
TP = 4
TT = TP * B
NTILES = POS_PER_W // TP
NT2 = NTILES // 2
EPS = 1e-12


def _rsqrt_newton(x):
    i = lax.bitcast_convert_type(x, jnp.int32)
    i = jnp.int32(0x5F3759DF) - lax.shift_right_logical(i, 1)
    y = lax.bitcast_convert_type(i, jnp.float32)
    for _ in range(4):
        y = y * (jnp.float32(1.5) - jnp.float32(0.5) * x * y * y)
    return y


def _lane_sum(x):
    idx0 = lax.iota(jnp.int32, L)
    for sh in (8, 4, 2, 1):
        x = x + x.at[idx0 ^ sh].get(mode="promise_in_bounds")
    return x


def _body(ids2, sids2, pids2, word, shape_t, pron, pos, tt, gamma, beta,
          out,
          idxw, idxs, idxp, oidx0, oidx1,
          wv0, sv0, pv0, wv1, sv1, pv1, outv0, outv1, posv0, posv1,
          ttv, gv, bv,
          semw0, sems0, semp0, semw1, sems1, semp1,
          semo0, semo1, semq0, semq1):
    cid = lax.axis_index("c")
    sid = lax.axis_index("s")
    wid = sid * NC + cid
    pos0 = wid * POS_PER_W
    third = jnp.float32(1.0 / 3.0)

    pltpu.sync_copy(tt.at[0], ttv)
    pltpu.sync_copy(gamma, gv)
    pltpu.sync_copy(beta, bv)
    row0 = wid * NTILES
    pltpu.sync_copy(ids2.at[pl.ds(row0, NTILES)], idxw)
    pltpu.sync_copy(sids2.at[pl.ds(row0, NTILES)], idxs)
    pltpu.sync_copy(pids2.at[pl.ds(row0, NTILES)], idxp)
    for i in range(NCH):
        bv[pl.ds(i * L, L)] = bv[pl.ds(i * L, L)] * third

    bufs = (
        (posv0, wv0, sv0, pv0, outv0, oidx0, semw0, sems0, semp0, semo0, semq0),
        (posv1, wv1, sv1, pv1, outv1, oidx1, semw1, sems1, semp1, semo1, semq1),
    )

    def launch(t, bf):
        posv, wv, sv, pv, outv, oidx, semw, sems, semp, semo, semq = bf
        pbase = pos0 + t * TP
        pltpu.async_copy(pos.at[pl.ds(pbase, TP)], posv, semq)
        pltpu.async_copy(word.at[idxw.at[t]], wv, semw)
        pltpu.async_copy(shape_t.at[idxs.at[t]], sv, sems)
        pltpu.async_copy(pron.at[idxp.at[t]], pv, semp)

    def finish(t, bf):
        posv, wv, sv, pv, outv, oidx, semw, sems, semp, semo, semq = bf
        pbase = pos0 + t * TP

        @pl.when(t >= 2)
        def _():
            pltpu.make_async_copy(outv, out.at[oidx], semo).wait()

        pltpu.make_async_copy(pos.at[pl.ds(pbase, TP)], posv, semq).wait()

        def fold(jp, _):
            for i in range(NCH):
                d = pl.ds(i * L, L)
                posv[jp, d] = posv[jp, d] + ttv[d]
            return 0
        lax.fori_loop(0, TP, fold, 0)
        k = lax.iota(jnp.int32, L)
        oidx[...] = (k & jnp.int32(B - 1)) * jnp.int32(S) \
            + pbase + lax.shift_right_logical(k, 2)

        pltpu.make_async_copy(word.at[idxw.at[t]], wv, semw).wait()
        pltpu.make_async_copy(shape_t.at[idxs.at[t]], sv, sems).wait()
        pltpu.make_async_copy(pron.at[idxp.at[t]], pv, semp).wait()

        z = jnp.zeros((L,), jnp.float32)

        HG = TT // 2
        lanes = lax.iota(jnp.int32, L)
        s_v, q_v = z, z
        for half in range(2):
            toks = list(range(half * HG, (half + 1) * HG))
            jps = sorted({j >> 2 for j in toks})

            def ch_acc(i, carry, toks=toks, jps=jps):
                d = pl.ds(i * L, L)
                pt = {jp: posv[jp, d] for jp in jps}
                acc = []
                for n, j in enumerate(toks):
                    v = wv[j, d] + pt[j >> 2]
                    acc.append(carry[n] + v)
                    acc.append(carry[HG + n] + v * v)
                return tuple(acc[0::2] + acc[1::2])

            accs = lax.fori_loop(0, NCH, ch_acc, tuple([z] * (2 * HG)))

            for n, j in enumerate(toks):
                sel = lanes == jnp.int32(j)
                s_v = jnp.where(sel, _lane_sum(accs[n]), s_v)
                q_v = jnp.where(sel, _lane_sum(accs[HG + n]), q_v)

        mean_v = s_v * jnp.float32(1.0 / H)
        var_v = q_v * jnp.float32(1.0 / H) - mean_v * mean_v
        a3_v = _rsqrt_newton(var_v + jnp.float32(EPS)) * third

        for half in range(2):
            toks = list(range(half * HG, (half + 1) * HG))
            jps = sorted({j >> 2 for j in toks})
            means = {}
            a3s = {}
            for j in toks:
                jsplat = jnp.full((L,), j, jnp.int32)
                means[j] = mean_v.at[jsplat].get(mode="promise_in_bounds")
                a3s[j] = a3_v.at[jsplat].get(mode="promise_in_bounds")

            def ch_out(i, _, toks=toks, jps=jps, means=means, a3s=a3s):
                d = pl.ds(i * L, L)
                g_ = gv[d]
                b3_ = bv[d]
                pt = {jp: posv[jp, d] for jp in jps}
                for j in toks:
                    v = wv[j, d] + pt[j >> 2]
                    o = (v - means[j]) * a3s[j] * g_ \
                        + (b3_ + (sv[j, d] + pv[j, d]) * third)
                    outv[j, d] = o
                return 0
            lax.fori_loop(0, NCH, ch_out, 0)

        pltpu.async_copy(outv, out.at[oidx], semo)

    launch(0, bufs[0])

    def pipe(i, _):
        t0 = 2 * i
        launch(t0 + 1, bufs[1])
        finish(t0, bufs[0])

        @pl.when(i < NT2 - 1)
        def _():
            launch(t0 + 2, bufs[0])

        finish(t0 + 1, bufs[1])
        return 0

    lax.fori_loop(0, NT2, pipe, 0)

    pltpu.make_async_copy(outv0, out.at[oidx0], semo0).wait()
    pltpu.make_async_copy(outv1, out.at[oidx1], semo1).wait()


@jax.jit
def _sc_call(ids_t, sids_t, pids_t, word, shape_t, pron, pos, tt, gamma, beta):
    mesh = plsc.VectorSubcoreMesh(core_axis_name="c", subcore_axis_name="s",
                                  num_cores=NC, num_subcores=NS)
    f = pl.kernel(
        _body,
        out_type=jax.ShapeDtypeStruct((B * S, H), jnp.float32),
        mesh=mesh,
        scratch_types=[
            pltpu.VMEM((NTILES, TT), jnp.int32),
            pltpu.VMEM((NTILES, TT), jnp.int32),
            pltpu.VMEM((NTILES, TT), jnp.int32),
            pltpu.VMEM((TT,), jnp.int32),
            pltpu.VMEM((TT,), jnp.int32),
            pltpu.VMEM((TT, H), jnp.float32),
            pltpu.VMEM((TT, H), jnp.float32),
            pltpu.VMEM((TT, H), jnp.float32),
            pltpu.VMEM((TT, H), jnp.float32),
            pltpu.VMEM((TT, H), jnp.float32),
            pltpu.VMEM((TT, H), jnp.float32),
            pltpu.VMEM((TT, H), jnp.float32),
            pltpu.VMEM((TT, H), jnp.float32),
            pltpu.VMEM((TP, H), jnp.float32),
            pltpu.VMEM((TP, H), jnp.float32),
            pltpu.VMEM((H,), jnp.float32),
            pltpu.VMEM((H,), jnp.float32),
            pltpu.VMEM((H,), jnp.float32),
            pltpu.SemaphoreType.DMA,
            pltpu.SemaphoreType.DMA,
            pltpu.SemaphoreType.DMA,
            pltpu.SemaphoreType.DMA,
            pltpu.SemaphoreType.DMA,
            pltpu.SemaphoreType.DMA,
            pltpu.SemaphoreType.DMA,
            pltpu.SemaphoreType.DMA,
            pltpu.SemaphoreType.DMA,
            pltpu.SemaphoreType.DMA,
        ],
    )
    return f(ids_t, sids_t, pids_t, word, shape_t, pron, pos, tt, gamma, beta)


def kernel(input_ids, input_shape_ids, input_pronunciation_ids,
           word_embeddings, shape_embed, pronunciation_embed,
           position_embeddings, token_type_embeddings, ln_weight, ln_bias):
    ids_t = input_ids.astype(jnp.int32).T.reshape(-1, TT)
    sids_t = input_shape_ids.astype(jnp.int32).T.reshape(-1, TT)
    pids_t = input_pronunciation_ids.astype(jnp.int32).T.reshape(-1, TT)
    out = _sc_call(ids_t, sids_t, pids_t, word_embeddings, shape_embed,
                   pronunciation_embed, position_embeddings,
                   token_type_embeddings, ln_weight, ln_bias)
    return out.reshape(B, S, H)
